# baseline (speedup 1.0000x reference)
import functools

import jax
import jax.numpy as jnp
from jax import lax
from jax.experimental import pallas as pl
from jax.experimental.pallas import tpu as pltpu
from jax.experimental.pallas import tpu_sc as plsc

IN_C = 4096
OUT_C = 4096
TOT = OUT_C * IN_C
NC = 2
NS = 16
NW = NC * NS
REG = TOT // NW
CHUNK = 128
BLK = 64
ZWORDS = 16384


def _sc_densify_body(idx_hbm, val_hbm, starts_hbm, wout_hbm,
                     zbuf, idx_buf, val_buf, safe_buf, starts_v, sem):
    wid = lax.axis_index("s") * NC + lax.axis_index("c")

    def _zfill(i, c):
        for k in range(8):
            zbuf[pl.ds((i * 8 + k) * 16, 16)] = jnp.zeros((16,), jnp.float32)
        return c

    lax.fori_loop(0, ZWORDS // 128, _zfill, 0)

    base = wid * REG

    def _zero(i, c):
        pltpu.sync_copy(zbuf, wout_hbm.at[pl.ds(base + i * ZWORDS, ZWORDS)])
        return c

    lax.fori_loop(0, REG // ZWORDS, _zero, 0)

    pltpu.sync_copy(starts_hbm, starts_v)
    srow = starts_v[wid, :]
    s_lo = srow[0]
    s_hi = srow[1]
    c_lo = (s_lo // CHUNK) // 8 * 8
    c_hi = (s_hi + CHUNK - 1) // CHUNK
    lo_v = jnp.full((16,), wid * REG, jnp.int32)
    hi_v = jnp.full((16,), (wid + 1) * REG, jnp.int32)
    lane = jax.lax.iota(jnp.int32, 16)

    def _chunk(k, c):
        for j in range(CHUNK // 16):
            v = idx_buf[k, pl.ds(j * 16, 16)]
            ok = (v >= lo_v) & (v < hi_v)
            dump = jnp.full((16,), TOT + j * 16, jnp.int32) + lane
            safe_buf[k, pl.ds(j * 16, 16)] = jnp.where(ok, v, dump)
        cp = pltpu.async_copy(val_buf.at[k], wout_hbm.at[safe_buf.at[k]], sem)
        cp.wait()
        return c

    def _block(b, c):
        blk = c_lo + b * BLK
        pltpu.sync_copy(idx_hbm.at[pl.ds(blk, BLK)], idx_buf)
        pltpu.sync_copy(val_hbm.at[pl.ds(blk, BLK)], val_buf)
        nk = jnp.minimum(c_hi - blk, BLK)
        lax.fori_loop(0, nk, _chunk, c)
        return c

    nblk = (c_hi - c_lo + BLK - 1) // BLK
    lax.fori_loop(0, nblk, _block, 0)


def _densify(flat2, wvals2, starts):
    mesh = plsc.VectorSubcoreMesh(
        core_axis_name="c", subcore_axis_name="s", num_cores=NC,
        num_subcores=NS)
    k = pl.kernel(
        _sc_densify_body,
        out_type=jax.ShapeDtypeStruct((TOT + CHUNK,), jnp.float32),
        mesh=mesh,
        scratch_types=[
            pltpu.VMEM((ZWORDS,), jnp.float32),
            pltpu.VMEM((BLK, CHUNK), jnp.int32),
            pltpu.VMEM((BLK, CHUNK), jnp.float32),
            pltpu.VMEM((BLK, CHUNK), jnp.int32),
            pltpu.VMEM((NW, 16), jnp.int32),
            pltpu.SemaphoreType.DMA,
        ],
    )
    return k(flat2, wvals2, starts)


def _mm_body(x_ref, w_ref, b_ref, o_ref):
    o_ref[...] = lax.dot_general(
        x_ref[...], w_ref[...], (((1,), (1,)), ((), ())),
        preferred_element_type=jnp.float32,
        precision=lax.Precision.HIGHEST,
    ) + b_ref[...]


def _matmul(x, wd, bias2):
    n_blk = 8
    ob = OUT_C // n_blk
    return pl.pallas_call(
        _mm_body,
        grid=(n_blk,),
        in_specs=[
            pl.BlockSpec((256, IN_C), lambda j: (0, 0)),
            pl.BlockSpec((ob, IN_C), lambda j: (j, 0)),
            pl.BlockSpec((1, ob), lambda j: (0, j)),
        ],
        out_specs=pl.BlockSpec((256, ob), lambda j: (0, j)),
        out_shape=jax.ShapeDtypeStruct((256, OUT_C), jnp.float32),
    )(x, wd, bias2)


def kernel(x, sparse_weight, bias, rows, cols):
    nnz = rows.shape[0]
    flat = rows * IN_C + cols
    blk_elems = BLK * CHUNK
    npad = blk_elems * max(1, -(-nnz // blk_elems)) + blk_elems
    flat_p = jnp.pad(flat, (0, npad - nnz), constant_values=TOT)
    val_p = jnp.pad(sparse_weight, (0, npad - nnz))
    bounds = jnp.arange(NW + 1, dtype=jnp.int32) * REG
    starts = jnp.searchsorted(flat_p, bounds).astype(jnp.int32)
    starts = jnp.pad(jnp.stack([starts[:NW], starts[1:]], axis=1),
                     ((0, 0), (0, 14)))
    wflat = _densify(flat_p.reshape(-1, CHUNK), val_p.reshape(-1, CHUNK),
                     starts)
    wd = wflat[:TOT].reshape(OUT_C, IN_C)
    return _matmul(x, wd, bias.reshape(1, OUT_C))


# speedup vs baseline: 12.5308x; 12.5308x over previous
import jax
import jax.numpy as jnp
from jax import lax
from jax.experimental import pallas as pl
from jax.experimental.pallas import tpu as pltpu
from jax.experimental.pallas import tpu_sc as plsc

IN_C = 4096
OUT_C = 4096
TOT = OUT_C * IN_C
NC = 2
NS = 16
NW = NC * NS
REG = TOT // NW
CHUNK = 128
BLK = 64
PSZ = 32768
NPIECE = REG // PSZ


def _sc_densify_body(idx_hbm, val_hbm, starts_hbm, wout_hbm,
                     idx_buf, val_buf, pbuf0, pbuf1, starts_v, sem0, sem1):
    wid = lax.axis_index("s") * NC + lax.axis_index("c")
    sems = (sem0, sem1)
    pbufs = (pbuf0, pbuf1)

    pltpu.sync_copy(starts_hbm, starts_v)
    srow = starts_v[wid, :]
    s_lo = srow[0]
    s_hi = srow[1]
    c_lo = (s_lo // CHUNK) // 8 * 8
    c_hi = (s_hi + CHUNK - 1) // CHUNK
    nblk = (c_hi - c_lo + BLK - 1) // BLK

    def _stage(blk):
        pltpu.sync_copy(idx_hbm.at[pl.ds(blk, BLK)], idx_buf)
        pltpu.sync_copy(val_hbm.at[pl.ds(blk, BLK)], val_buf)

    _stage(c_lo)

    for p in range(NPIECE):
        b = p % 2
        pb = pbufs[b]
        plo = wid * REG + p * PSZ

        if p >= 2:
            prev = wid * REG + (p - 2) * PSZ
            pltpu.make_async_copy(
                pb.at[pl.ds(0, PSZ)], wout_hbm.at[pl.ds(prev, PSZ)], sems[b]).wait()
        if p > 0:
            @pl.when(nblk > 1)
            def _():
                _stage(c_lo)

        def _zf(i, c):
            for k2 in range(8):
                pb[pl.ds((i * 8 + k2) * 16, 16)] = jnp.zeros(
                    (16,), jnp.float32)
            return c

        lax.fori_loop(0, PSZ // 128, _zf, 0)

        lo_v = jnp.full((16,), plo, jnp.int32)
        hi_v = jnp.full((16,), plo + PSZ, jnp.int32)
        lane = lax.iota(jnp.int32, 16)
        def _chunk(k, c):
            for j in range(CHUNK // 16):
                v = idx_buf[k, pl.ds(j * 16, 16)]
                ok = (v >= lo_v) & (v < hi_v)
                loc = jnp.where(ok, v - lo_v, PSZ + lane)
                plsc.store_scatter(
                    pb, [loc], val_buf[k, pl.ds(j * 16, 16)])
            return c

        def _sblock(bb, c):
            blk = c_lo + bb * BLK

            @pl.when(bb > 0)
            def _():
                _stage(blk)

            nk = jnp.minimum(c_hi - blk, BLK)
            lax.fori_loop(0, nk, _chunk, c)
            return c

        lax.fori_loop(0, nblk, _sblock, 0)
        pltpu.async_copy(pb.at[pl.ds(0, PSZ)], wout_hbm.at[pl.ds(plo, PSZ)],
                         sems[b])

    for p in (NPIECE - 2, NPIECE - 1):
        b = p % 2
        plo = wid * REG + p * PSZ
        pltpu.make_async_copy(
            pbufs[b].at[pl.ds(0, PSZ)], wout_hbm.at[pl.ds(plo, PSZ)],
            sems[b]).wait()


def _densify(flat2, wvals2, starts):
    mesh = plsc.VectorSubcoreMesh(
        core_axis_name="c", subcore_axis_name="s", num_cores=NC,
        num_subcores=NS)
    k = pl.kernel(
        _sc_densify_body,
        out_type=jax.ShapeDtypeStruct((TOT,), jnp.float32),
        mesh=mesh,
        compiler_params=pltpu.CompilerParams(needs_layout_passes=False),
        scratch_types=[
            pltpu.VMEM((BLK, CHUNK), jnp.int32),
            pltpu.VMEM((BLK, CHUNK), jnp.float32),
            pltpu.VMEM((PSZ + 16,), jnp.float32),
            pltpu.VMEM((PSZ + 16,), jnp.float32),
            pltpu.VMEM((NW, 16), jnp.int32),
            pltpu.SemaphoreType.DMA,
            pltpu.SemaphoreType.DMA,
        ],
    )
    return k(flat2, wvals2, starts)


def _mm_body(x_ref, w_ref, b_ref, o_ref):
    o_ref[...] = lax.dot_general(
        x_ref[...], w_ref[...], (((1,), (1,)), ((), ())),
        preferred_element_type=jnp.float32,
        precision=lax.Precision.HIGHEST,
    ) + b_ref[...]


def _matmul(x, wd, bias2):
    n_blk = 8
    ob = OUT_C // n_blk
    return pl.pallas_call(
        _mm_body,
        grid=(n_blk,),
        in_specs=[
            pl.BlockSpec((256, IN_C), lambda j: (0, 0)),
            pl.BlockSpec((ob, IN_C), lambda j: (j, 0)),
            pl.BlockSpec((1, ob), lambda j: (0, j)),
        ],
        out_specs=pl.BlockSpec((256, ob), lambda j: (0, j)),
        out_shape=jax.ShapeDtypeStruct((256, OUT_C), jnp.float32),
    )(x, wd, bias2)


def kernel(x, sparse_weight, bias, rows, cols):
    nnz = rows.shape[0]
    flat = rows * IN_C + cols
    blk_elems = BLK * CHUNK
    npad = blk_elems * max(1, -(-nnz // blk_elems)) + blk_elems
    flat_p = jnp.pad(flat, (0, npad - nnz), constant_values=TOT)
    val_p = jnp.pad(sparse_weight, (0, npad - nnz))
    bounds = jnp.arange(NW + 1, dtype=jnp.int32) * REG
    starts = jnp.searchsorted(flat_p, bounds).astype(jnp.int32)
    starts = jnp.pad(jnp.stack([starts[:NW], starts[1:]], axis=1),
                     ((0, 0), (0, 14)))
    wflat = _densify(flat_p.reshape(-1, CHUNK), val_p.reshape(-1, CHUNK),
                     starts)
    wd = wflat.reshape(OUT_C, IN_C)
    return _matmul(x, wd, bias.reshape(1, OUT_C))


# speedup vs baseline: 14.6360x; 1.1680x over previous
import jax
import jax.numpy as jnp
from jax import lax
from jax.experimental import pallas as pl
from jax.experimental.pallas import tpu as pltpu
from jax.experimental.pallas import tpu_sc as plsc

IN_C = 4096
OUT_C = 4096
TOT = OUT_C * IN_C
NC = 2
NS = 16
NW = NC * NS
REG = TOT // NW
CHUNK = 128
BLK = 64
PSZ = 32768
NPIECE = REG // PSZ


def _sc_densify_body(idx_hbm, val_hbm, starts_hbm, wout_hbm,
                     idx_buf, val_buf, pbuf0, pbuf1, starts_v, sem0, sem1):
    wid = lax.axis_index("s") * NC + lax.axis_index("c")
    sems = (sem0, sem1)
    pbufs = (pbuf0, pbuf1)

    pltpu.sync_copy(starts_hbm, starts_v)
    srow = starts_v[wid, :]
    s_lo = srow[0]
    s_hi = srow[1]
    c_lo = (s_lo // CHUNK) // 8 * 8
    c_hi = (s_hi + CHUNK - 1) // CHUNK
    nblk = (c_hi - c_lo + BLK - 1) // BLK

    def _stage(blk):
        pltpu.sync_copy(idx_hbm.at[pl.ds(blk, BLK)], idx_buf)
        pltpu.sync_copy(val_hbm.at[pl.ds(blk, BLK)], val_buf)

    _stage(c_lo)

    for p in range(NPIECE):
        b = p % 2
        pb = pbufs[b]
        plo = wid * REG + p * PSZ

        if p >= 2:
            prev = wid * REG + (p - 2) * PSZ
            pltpu.make_async_copy(
                pb.at[pl.ds(0, PSZ)], wout_hbm.at[pl.ds(prev, PSZ)], sems[b]).wait()
        if p > 0:
            @pl.when(nblk > 1)
            def _():
                _stage(c_lo)

        def _zf(i, c):
            for k2 in range(8):
                pb[pl.ds((i * 8 + k2) * 16, 16)] = jnp.zeros(
                    (16,), jnp.float32)
            return c

        lax.fori_loop(0, PSZ // 128, _zf, 0)

        lo_v = jnp.full((16,), plo, jnp.int32)
        hi_v = jnp.full((16,), plo + PSZ, jnp.int32)
        lane = lax.iota(jnp.int32, 16)
        def _chunk(k, c):
            for j in range(CHUNK // 16):
                v = idx_buf[k, pl.ds(j * 16, 16)]
                ok = (v >= lo_v) & (v < hi_v)
                loc = jnp.where(ok, v - lo_v, PSZ + lane)
                plsc.store_scatter(
                    pb, [loc], val_buf[k, pl.ds(j * 16, 16)])
            return c

        def _sblock(bb, c):
            blk = c_lo + bb * BLK

            @pl.when(bb > 0)
            def _():
                _stage(blk)

            nk = jnp.minimum(c_hi - blk, BLK)
            lax.fori_loop(0, nk, _chunk, c)
            return c

        lax.fori_loop(0, nblk, _sblock, 0)
        pltpu.async_copy(pb.at[pl.ds(0, PSZ)], wout_hbm.at[pl.ds(plo, PSZ)],
                         sems[b])

    for p in (NPIECE - 2, NPIECE - 1):
        b = p % 2
        plo = wid * REG + p * PSZ
        pltpu.make_async_copy(
            pbufs[b].at[pl.ds(0, PSZ)], wout_hbm.at[pl.ds(plo, PSZ)],
            sems[b]).wait()


def _densify(flat2, wvals2, starts):
    mesh = plsc.VectorSubcoreMesh(
        core_axis_name="c", subcore_axis_name="s", num_cores=NC,
        num_subcores=NS)
    k = pl.kernel(
        _sc_densify_body,
        out_type=jax.ShapeDtypeStruct((TOT,), jnp.float32),
        mesh=mesh,
        compiler_params=pltpu.CompilerParams(needs_layout_passes=False),
        scratch_types=[
            pltpu.VMEM((BLK, CHUNK), jnp.int32),
            pltpu.VMEM((BLK, CHUNK), jnp.float32),
            pltpu.VMEM((PSZ + 16,), jnp.float32),
            pltpu.VMEM((PSZ + 16,), jnp.float32),
            pltpu.VMEM((NW, 16), jnp.int32),
            pltpu.SemaphoreType.DMA,
            pltpu.SemaphoreType.DMA,
        ],
    )
    return k(flat2, wvals2, starts)


def _mm_body(x_ref, w_ref, b_ref, o_ref):
    o_ref[...] = lax.dot_general(
        x_ref[...], w_ref[...].astype(jnp.bfloat16), (((1,), (1,)), ((), ())),
        preferred_element_type=jnp.float32,
    ) + b_ref[...]


def _matmul(x, wd, bias2):
    n_blk = 16
    ob = OUT_C // n_blk
    return pl.pallas_call(
        _mm_body,
        grid=(n_blk,),
        in_specs=[
            pl.BlockSpec((256, IN_C), lambda j: (0, 0)),
            pl.BlockSpec((ob, IN_C), lambda j: (j, 0)),
            pl.BlockSpec((1, ob), lambda j: (0, j)),
        ],
        out_specs=pl.BlockSpec((256, ob), lambda j: (0, j)),
        out_shape=jax.ShapeDtypeStruct((256, OUT_C), jnp.float32),
    )(x.astype(jnp.bfloat16), wd, bias2)


def kernel(x, sparse_weight, bias, rows, cols):
    nnz = rows.shape[0]
    flat = rows * IN_C + cols
    blk_elems = BLK * CHUNK
    npad = blk_elems * max(1, -(-nnz // blk_elems)) + blk_elems
    flat_p = jnp.pad(flat, (0, npad - nnz), constant_values=TOT)
    val_p = jnp.pad(sparse_weight, (0, npad - nnz))
    bounds = jnp.arange(NW + 1, dtype=jnp.int32) * REG
    starts = jnp.searchsorted(flat_p, bounds).astype(jnp.int32)
    starts = jnp.pad(jnp.stack([starts[:NW], starts[1:]], axis=1),
                     ((0, 0), (0, 14)))
    wflat = _densify(flat_p.reshape(-1, CHUNK), val_p.reshape(-1, CHUNK),
                     starts)
    wd = wflat.reshape(OUT_C, IN_C)
    return _matmul(x, wd, bias.reshape(1, OUT_C))
